# baseline (speedup 1.0000x reference)
import jax
import jax.numpy as jnp
from jax.experimental import pallas as pl
from jax.experimental.pallas import tpu as pltpu

_B, _SEG, _D, _H, _O = 16, 2048, 1024, 256, 32


def _readout_body(x_ref, w1_ref, w2_ref, o_ref, xb_ref):
    i = pl.program_id(0)
    cur = jax.lax.rem(i, 2)
    xb_ref[pl.ds(cur * _SEG, _SEG), :] = x_ref[...].astype(jnp.bfloat16)

    @pl.when(i == 0)
    def _prologue():
        o_ref[...] = jnp.zeros_like(o_ref)

    @pl.when(i > 0)
    def _attend():
        prev = 1 - cur
        xb = xb_ref[pl.ds(prev * _SEG, _SEG), :]
        w2 = w2_ref[...]
        t = jnp.tanh(jnp.dot(xb, w1_ref[...].astype(jnp.bfloat16).T,
                             preferred_element_type=jnp.float32))
        s = jnp.dot(t.astype(jnp.bfloat16), w2.astype(jnp.bfloat16).T,
                    preferred_element_type=jnp.float32)
        k = jnp.sum(jnp.abs(w2), axis=1)
        e = jnp.exp(s - k[None, :])
        l = jnp.sum(e, axis=0)
        acc = jax.lax.dot_general(
            e.astype(jnp.bfloat16), xb, (((0,), (0,)), ((), ())),
            preferred_element_type=jnp.float32)
        o_ref[...] = acc / l[:, None]


def kernel(embeddings, scope, w1, w2):
    del scope
    out = pl.pallas_call(
        _readout_body,
        grid=(_B + 1,),
        in_specs=[
            pl.BlockSpec((_SEG, _D), lambda i: (jnp.minimum(i, _B - 1), 0)),
            pl.BlockSpec((_H, _D), lambda i: (0, 0)),
            pl.BlockSpec((_O, _H), lambda i: (0, 0)),
        ],
        out_specs=pl.BlockSpec((_O, _D), lambda i: (jnp.maximum(i - 1, 0), 0)),
        out_shape=jax.ShapeDtypeStruct((_B * _O, _D), jnp.float32),
        scratch_shapes=[pltpu.VMEM((2 * _SEG, _D), jnp.bfloat16)],
    )(embeddings, w1, w2)
    return out.reshape(_B, _O * _D)


# speedup vs baseline: 1.0228x; 1.0228x over previous
import jax
import jax.numpy as jnp
from jax.experimental import pallas as pl
from jax.experimental.pallas import tpu as pltpu

_B, _SEG, _D, _H, _O = 16, 2048, 1024, 256, 32


def _readout_body(x_ref, w1_ref, w2_ref, o_ref, xb_ref):
    i = pl.program_id(0)
    cur = jax.lax.rem(i, 2)
    xb_ref[pl.ds(cur * _SEG, _SEG), :] = x_ref[...].astype(jnp.bfloat16)

    prev = 1 - cur
    xb = xb_ref[pl.ds(prev * _SEG, _SEG), :]
    w2 = w2_ref[...]
    t = jnp.tanh(jnp.dot(xb, w1_ref[...].astype(jnp.bfloat16).T,
                         preferred_element_type=jnp.float32))
    s = jnp.dot(t.astype(jnp.bfloat16), w2.astype(jnp.bfloat16).T,
                preferred_element_type=jnp.float32)
    k = jnp.sum(jnp.abs(w2), axis=1)
    e = jnp.exp(s - k[None, :])
    l = jnp.sum(e, axis=0)
    acc = jax.lax.dot_general(
        e.astype(jnp.bfloat16), xb, (((0,), (0,)), ((), ())),
        preferred_element_type=jnp.float32)
    o_ref[...] = acc / l[:, None]


def kernel(embeddings, scope, w1, w2):
    del scope
    out = pl.pallas_call(
        _readout_body,
        grid=(_B + 1,),
        in_specs=[
            pl.BlockSpec((_SEG, _D), lambda i: (jnp.minimum(i, _B - 1), 0)),
            pl.BlockSpec((_H, _D), lambda i: (0, 0)),
            pl.BlockSpec((_O, _H), lambda i: (0, 0)),
        ],
        out_specs=pl.BlockSpec((_O, _D), lambda i: (jnp.maximum(i - 1, 0), 0)),
        out_shape=jax.ShapeDtypeStruct((_B * _O, _D), jnp.float32),
        scratch_shapes=[pltpu.VMEM((2 * _SEG, _D), jnp.bfloat16)],
    )(embeddings, w1, w2)
    return out.reshape(_B, _O * _D)
